# Initial kernel scaffold; baseline (speedup 1.0000x reference)
import jax
import jax.numpy as jnp
from jax.experimental import pallas as pl


def kernel(stem_w, stem_b, l1b0_c1_w, l1b0_c1_b, l1b0_c2_w, l1b0_c2_b, l1b0_c3_w, l1b0_c3_b, l1b0_down_w, l1b0_down_b, l1b1_c1_w, l1b1_c1_b, l1b1_c2_w, l1b1_c2_b, l1b1_c3_w, l1b1_c3_b, l1b2_c1_w, l1b2_c1_b, l1b2_c2_w, l1b2_c2_b, l1b2_c3_w, l1b2_c3_b, l2b0_c1_w, l2b0_c1_b, l2b0_c2_w, l2b0_c2_b, l2b0_c3_w, l2b0_c3_b, l2b0_down_w, l2b0_down_b, l2b1_c1_w, l2b1_c1_b, l2b1_c2_w, l2b1_c2_b, l2b1_c3_w, l2b1_c3_b, l2b2_c1_w, l2b2_c1_b, l2b2_c2_w, l2b2_c2_b, l2b2_c3_w, l2b2_c3_b, l2b3_c1_w, l2b3_c1_b, l2b3_c2_w, l2b3_c2_b, l2b3_c3_w, l2b3_c3_b, l3b0_c1_w, l3b0_c1_b, l3b0_c2_w, l3b0_c2_b, l3b0_c3_w, l3b0_c3_b, l3b0_down_w, l3b0_down_b, l3b1_c1_w, l3b1_c1_b, l3b1_c2_w, l3b1_c2_b, l3b1_c3_w, l3b1_c3_b, l3b2_c1_w, l3b2_c1_b, l3b2_c2_w, l3b2_c2_b, l3b2_c3_w, l3b2_c3_b, l3b3_c1_w, l3b3_c1_b, l3b3_c2_w, l3b3_c2_b, l3b3_c3_w, l3b3_c3_b, l3b4_c1_w, l3b4_c1_b, l3b4_c2_w, l3b4_c2_b, l3b4_c3_w, l3b4_c3_b, l3b5_c1_w, l3b5_c1_b, l3b5_c2_w, l3b5_c2_b, l3b5_c3_w, l3b5_c3_b, l4b0_c1_w, l4b0_c1_b, l4b0_c2_w, l4b0_c2_b, l4b0_c3_w, l4b0_c3_b, l4b0_down_w, l4b0_down_b, l4b1_c1_w, l4b1_c1_b, l4b1_c2_w, l4b1_c2_b, l4b1_c3_w, l4b1_c3_b, l4b2_c1_w, l4b2_c1_b, l4b2_c2_w, l4b2_c2_b, l4b2_c3_w, l4b2_c3_b, head_w, head_b, x):
    raise NotImplementedError("write your pallas kernel here")


# speedup vs baseline: 1.1592x; 1.1592x over previous
import functools

import jax
import jax.numpy as jnp
from jax.experimental import pallas as pl
from jax.experimental.pallas import tpu as pltpu


def _rup(x, m):
    return ((x + m - 1) // m) * m


def _np2(x):
    n = 1
    while n < x:
        n *= 2
    return n


_VMEM = 64 * 1024 * 1024


def _mm_body(*refs, relu, mode):
    if mode == "plain":
        x_ref, w_ref, b_ref, o_ref = refs
        y = jnp.dot(x_ref[...], w_ref[...], preferred_element_type=jnp.float32)
        y = y + b_ref[...]
    else:
        x_ref, w_ref, b_ref, r_ref, wd_ref, bd_ref, o_ref = refs
        y = jnp.dot(x_ref[...], w_ref[...], preferred_element_type=jnp.float32)
        y = y + b_ref[...]
        y = y + jnp.dot(r_ref[...], wd_ref[...], preferred_element_type=jnp.float32)
        y = y + bd_ref[...]
    if relu:
        y = jnp.maximum(y, 0.0)
    o_ref[...] = y.astype(o_ref.dtype)


def _matmul(x, w, b, relu, r=None, wd=None, bd=None):
    M, K = x.shape
    N = w.shape[1]
    tm = min(512, _rup(M, 16))
    Mp = _rup(M, tm)
    tn = min(512, N)
    if Mp != M:
        x = jnp.pad(x, ((0, Mp - M), (0, 0)))
        if r is not None:
            r = jnp.pad(r, ((0, Mp - M), (0, 0)))
    mode = "plain" if r is None else "proj"
    inputs = [x, w, b]
    specs = [
        pl.BlockSpec((tm, K), lambda i, j: (i, 0)),
        pl.BlockSpec((K, tn), lambda i, j: (0, j)),
        pl.BlockSpec((1, tn), lambda i, j: (0, j)),
    ]
    if r is not None:
        Kd = r.shape[1]
        inputs += [r, wd, bd]
        specs += [
            pl.BlockSpec((tm, Kd), lambda i, j: (i, 0)),
            pl.BlockSpec((Kd, tn), lambda i, j: (0, j)),
            pl.BlockSpec((1, tn), lambda i, j: (0, j)),
        ]
    out = pl.pallas_call(
        functools.partial(_mm_body, relu=relu, mode=mode),
        out_shape=jax.ShapeDtypeStruct((Mp, N), jnp.bfloat16),
        grid=(Mp // tm, N // tn),
        in_specs=specs,
        out_specs=pl.BlockSpec((tm, tn), lambda i, j: (i, j)),
        compiler_params=pltpu.CompilerParams(
            dimension_semantics=("parallel", "parallel"),
            vmem_limit_bytes=_VMEM),
    )(*inputs)
    return out if Mp == M else out[:M]


def _block_kernel(refs, *, wp, H, tm, Cm, down):
    if down:
        (xm_ref, xh_ref, mm_ref, mh_ref, w1_ref, b1_ref, w2_ref, b2_ref,
         w3_ref, b3_ref, wd_ref, bd_ref, o_ref, y1_ref) = refs
    else:
        (xm_ref, xh_ref, mm_ref, mh_ref, w1_ref, b1_ref, w2_ref, b2_ref,
         w3_ref, b3_ref, o_ref, y1_ref) = refs
        wd_ref = bd_ref = None
    y1m = jnp.dot(xm_ref[...], w1_ref[...], preferred_element_type=jnp.float32)
    y1m = jnp.maximum(y1m + b1_ref[...], 0.0) * mm_ref[...]
    y1_ref[0:tm, :] = y1m
    y1h = jnp.dot(xh_ref[...], w1_ref[...], preferred_element_type=jnp.float32)
    y1h = jnp.maximum(y1h + b1_ref[...], 0.0) * mh_ref[...]
    y1_ref[tm:, :] = y1h
    base = H - (wp + 1)
    acc = b2_ref[...].astype(jnp.float32)
    for t in range(9):
        off = base + (t // 3) * wp + (t % 3)
        tap = y1_ref[off:off + tm, :].astype(jnp.bfloat16)
        acc = acc + jnp.dot(tap, w2_ref[t * Cm:(t + 1) * Cm, :],
                            preferred_element_type=jnp.float32)
    y2 = jnp.maximum(acc, 0.0).astype(jnp.bfloat16)
    y3 = jnp.dot(y2, w3_ref[...], preferred_element_type=jnp.float32)
    y3 = y3 + b3_ref[...]
    res = jnp.concatenate([xm_ref[H:tm, :], xh_ref[0:H, :]], axis=0)
    if down:
        rp = jnp.dot(res, wd_ref[...], preferred_element_type=jnp.float32)
        y3 = y3 + rp + bd_ref[...]
    else:
        y3 = y3 + res.astype(jnp.float32)
    omask = jnp.concatenate([mm_ref[H:tm, :], mh_ref[0:H, :]], axis=0)
    o_ref[...] = (jnp.maximum(y3, 0.0) * omask).astype(o_ref.dtype)


def _fused_block(xflat, maskext, n, hp, wp, w1, b1, w2, b2, w3, b3,
                 wd=None, bd=None):
    M = n * hp * wp
    Cin = xflat.shape[1]
    Cm = w2.shape[1]
    Cout = w3.shape[1]
    H = max(16, _np2(wp + 2))
    tm = 256
    Mp = _rup(M, tm)
    xext = jnp.pad(xflat, ((H, Mp - M + H), (0, 0)))
    nhb = tm // (2 * H)

    refs_in = [xext, xext, maskext, maskext, w1, b1, w2, b2, w3, b3]
    specs = [
        pl.BlockSpec((tm, Cin), lambda i: (i, 0)),
        pl.BlockSpec((2 * H, Cin), lambda i: ((i + 1) * nhb, 0)),
        pl.BlockSpec((tm, 1), lambda i: (i, 0)),
        pl.BlockSpec((2 * H, 1), lambda i: ((i + 1) * nhb, 0)),
        pl.BlockSpec((Cin, Cm), lambda i: (0, 0)),
        pl.BlockSpec((1, Cm), lambda i: (0, 0)),
        pl.BlockSpec((9 * Cm, Cm), lambda i: (0, 0)),
        pl.BlockSpec((1, Cm), lambda i: (0, 0)),
        pl.BlockSpec((Cm, Cout), lambda i: (0, 0)),
        pl.BlockSpec((1, Cout), lambda i: (0, 0)),
    ]
    down = wd is not None
    if down:
        refs_in += [wd, bd]
        specs += [
            pl.BlockSpec((Cin, Cout), lambda i: (0, 0)),
            pl.BlockSpec((1, Cout), lambda i: (0, 0)),
        ]

    def body(*refs):
        _block_kernel(refs, wp=wp, H=H, tm=tm, Cm=Cm, down=down)

    out = pl.pallas_call(
        body,
        out_shape=jax.ShapeDtypeStruct((Mp, Cout), jnp.bfloat16),
        grid=(Mp // tm,),
        in_specs=specs,
        out_specs=pl.BlockSpec((tm, Cout), lambda i: (i, 0)),
        scratch_shapes=[pltpu.VMEM((tm + 2 * H, Cm), jnp.float32)],
        compiler_params=pltpu.CompilerParams(
            dimension_semantics=("parallel",),
            vmem_limit_bytes=_VMEM),
    )(*refs_in)
    return out[:M]


def _ring_mask(n, hp, wp, H, tm):
    M = n * hp * wp
    Mp = _rup(M, tm)
    r = jnp.arange(M, dtype=jnp.int32)
    p = r % (hp * wp)
    a = p // wp
    b = p % wp
    ok = (a >= 1) & (a < hp - 1) & (b >= 1) & (b < wp - 1)
    m = ok.astype(jnp.float32).reshape(M, 1)
    return jnp.pad(m, ((H, Mp - M + H), (0, 0)))


def _s2conv_body(zm_ref, zh_ref, w_ref, b_ref, o_ref, zbuf_ref, *, Wv, tm, C4):
    tm_, _ = zm_ref.shape
    zbuf_ref[0:tm, :] = zm_ref[...].astype(jnp.float32)
    zbuf_ref[tm:, :] = zh_ref[...].astype(jnp.float32)
    acc = b_ref[...].astype(jnp.float32)
    for t, off in enumerate((0, 1, Wv, Wv + 1)):
        tap = zbuf_ref[off:off + tm, :].astype(jnp.bfloat16)
        acc = acc + jnp.dot(tap, w_ref[t * C4:(t + 1) * C4, :],
                            preferred_element_type=jnp.float32)
    o_ref[...] = jnp.maximum(acc, 0.0).astype(o_ref.dtype)


def _conv3x3_s2(x, w, b):
    n, h, wdt, C = x.shape
    xpad = jnp.pad(x, ((0, 0), (1, 1), (1, 1), (0, 0)))
    Ha, Wv = (h + 2) // 2, (wdt + 2) // 2
    z = xpad.reshape(n, Ha, 2, Wv, 2, C)
    z = jnp.transpose(z, (0, 1, 3, 2, 4, 5)).reshape(n * Ha * Wv, 4 * C)
    M = n * Ha * Wv
    H = max(16, _np2(Wv + 2))
    tm = 256
    Mp = _rup(M, tm)
    zext = jnp.pad(z, ((0, Mp - M + H), (0, 0)))
    Cout = w.shape[1]
    out = pl.pallas_call(
        functools.partial(_s2conv_body, Wv=Wv, tm=tm, C4=4 * C),
        out_shape=jax.ShapeDtypeStruct((Mp, Cout), jnp.bfloat16),
        grid=(Mp // tm,),
        in_specs=[
            pl.BlockSpec((tm, 4 * C), lambda i: (i, 0)),
            pl.BlockSpec((H, 4 * C), lambda i: ((i + 1) * (tm // H), 0)),
            pl.BlockSpec((16 * C, Cout), lambda i: (0, 0)),
            pl.BlockSpec((1, Cout), lambda i: (0, 0)),
        ],
        out_specs=pl.BlockSpec((tm, Cout), lambda i: (i, 0)),
        scratch_shapes=[pltpu.VMEM((tm + H, 4 * C), jnp.float32)],
        compiler_params=pltpu.CompilerParams(
            dimension_semantics=("parallel",),
            vmem_limit_bytes=_VMEM),
    )(zext, zext, w, b)
    ho, wo = h // 2, wdt // 2
    return out[:M].reshape(n, Ha, Wv, Cout)[:, :ho, :wo, :]


def _maxpool_body(x_ref, o_ref, *, ho, wo, c):
    X = x_ref[0].astype(jnp.float32)
    hp = X.shape[0]
    E = X.reshape(hp // 2, 2, X.shape[1], 2 * c)
    ev = E[:, 0]
    od = E[:, 1]
    v = jnp.maximum(jnp.maximum(ev[0:ho], od[0:ho]), ev[1:ho + 1])
    m = jnp.maximum(jnp.maximum(v[:, 0:wo, 0:c], v[:, 0:wo, c:2 * c]),
                    v[:, 1:wo + 1, 0:c])
    o_ref[0] = m.astype(o_ref.dtype)


def _maxpool(x):
    n, h, w, c = x.shape
    ho, wo = (h + 2 - 3) // 2 + 1, (w + 2 - 3) // 2 + 1
    xp = jnp.pad(x, ((0, 0), (1, 1), (1, 1), (0, 0)))
    hp, wv = h + 2, (w + 2) // 2
    xv = xp.reshape(n, hp, wv, 2 * c)
    return pl.pallas_call(
        functools.partial(_maxpool_body, ho=ho, wo=wo, c=c),
        out_shape=jax.ShapeDtypeStruct((n, ho, wo, c), x.dtype),
        grid=(n,),
        in_specs=[pl.BlockSpec((1, hp, wv, 2 * c), lambda i: (i, 0, 0, 0))],
        out_specs=pl.BlockSpec((1, ho, wo, c), lambda i: (i, 0, 0, 0)),
        compiler_params=pltpu.CompilerParams(
            dimension_semantics=("parallel",),
            vmem_limit_bytes=_VMEM),
    )(xv)


def _gap_body(x_ref, o_ref, *, inv):
    o_ref[...] = (jnp.sum(x_ref[...].astype(jnp.float32), axis=1,
                          keepdims=True) * inv).astype(o_ref.dtype)


def _gap(x):
    n, h, w, c = x.shape
    x2 = x.reshape(n, h * w, c)
    out = pl.pallas_call(
        functools.partial(_gap_body, inv=1.0 / (h * w)),
        out_shape=jax.ShapeDtypeStruct((n, 1, c), x.dtype),
        grid=(n,),
        in_specs=[pl.BlockSpec((1, h * w, c), lambda i: (i, 0, 0))],
        out_specs=pl.BlockSpec((1, 1, c), lambda i: (i, 0, 0)),
        compiler_params=pltpu.CompilerParams(
            dimension_semantics=("parallel",),
            vmem_limit_bytes=_VMEM),
    )(x2)
    return out.reshape(n, c)


def _stem(x, w, b):
    n, h, wdt, c = x.shape
    xp = jnp.pad(x, ((0, 0), (3, 3), (3, 3), (0, 0)))
    ho = (h + 6 - 7) // 2 + 1
    wo = (wdt + 6 - 7) // 2 + 1
    cols = [xp[:, i:i + 2 * ho:2, j:j + 2 * wo:2, :]
            for i in range(7) for j in range(7)]
    patches = jnp.stack(cols, axis=3).reshape(n * ho * wo, 49 * c)
    Kp = w.shape[0]
    patches = jnp.pad(patches, ((0, 0), (0, Kp - 49 * c)))
    out = _matmul(patches, w, b, relu=True)
    return out.reshape(n, ho, wo, -1)


_CFG = [(3, 1), (4, 2), (6, 2), (3, 2)]


def _pad_flat(x, cpad):
    n, h, w, c = x.shape
    xp = jnp.pad(x, ((0, 0), (1, 1), (1, 1), (0, cpad - c)))
    return xp.reshape(n * (h + 2) * (w + 2), cpad), h + 2, w + 2


def _unflat(xflat, n, hp, wp):
    return xflat.reshape(n, hp, wp, -1)[:, 1:hp - 1, 1:wp - 1, :]


def kernel(stem_w, stem_b, l1b0_c1_w, l1b0_c1_b, l1b0_c2_w, l1b0_c2_b, l1b0_c3_w, l1b0_c3_b, l1b0_down_w, l1b0_down_b, l1b1_c1_w, l1b1_c1_b, l1b1_c2_w, l1b1_c2_b, l1b1_c3_w, l1b1_c3_b, l1b2_c1_w, l1b2_c1_b, l1b2_c2_w, l1b2_c2_b, l1b2_c3_w, l1b2_c3_b, l2b0_c1_w, l2b0_c1_b, l2b0_c2_w, l2b0_c2_b, l2b0_c3_w, l2b0_c3_b, l2b0_down_w, l2b0_down_b, l2b1_c1_w, l2b1_c1_b, l2b1_c2_w, l2b1_c2_b, l2b1_c3_w, l2b1_c3_b, l2b2_c1_w, l2b2_c1_b, l2b2_c2_w, l2b2_c2_b, l2b2_c3_w, l2b2_c3_b, l2b3_c1_w, l2b3_c1_b, l2b3_c2_w, l2b3_c2_b, l2b3_c3_w, l2b3_c3_b, l3b0_c1_w, l3b0_c1_b, l3b0_c2_w, l3b0_c2_b, l3b0_c3_w, l3b0_c3_b, l3b0_down_w, l3b0_down_b, l3b1_c1_w, l3b1_c1_b, l3b1_c2_w, l3b1_c2_b, l3b1_c3_w, l3b1_c3_b, l3b2_c1_w, l3b2_c1_b, l3b2_c2_w, l3b2_c2_b, l3b2_c3_w, l3b2_c3_b, l3b3_c1_w, l3b3_c1_b, l3b3_c2_w, l3b3_c2_b, l3b3_c3_w, l3b3_c3_b, l3b4_c1_w, l3b4_c1_b, l3b4_c2_w, l3b4_c2_b, l3b4_c3_w, l3b4_c3_b, l3b5_c1_w, l3b5_c1_b, l3b5_c2_w, l3b5_c2_b, l3b5_c3_w, l3b5_c3_b, l4b0_c1_w, l4b0_c1_b, l4b0_c2_w, l4b0_c2_b, l4b0_c3_w, l4b0_c3_b, l4b0_down_w, l4b0_down_b, l4b1_c1_w, l4b1_c1_b, l4b1_c2_w, l4b1_c2_b, l4b1_c3_w, l4b1_c3_b, l4b2_c1_w, l4b2_c1_b, l4b2_c2_w, l4b2_c2_b, l4b2_c3_w, l4b2_c3_b, head_w, head_b, x):
    P = locals()
    n = x.shape[0]
    h = jnp.transpose(x, (0, 2, 3, 1)).astype(jnp.bfloat16)
    h = _stem(h, stem_w, stem_b)
    h = _maxpool(h)

    for li, (blocks, stride) in enumerate(_CFG):
        ln = li + 1
        if stride == 2:
            nb, hh, ww, C = h.shape
            y1 = _matmul(h.reshape(nb * hh * ww, C),
                         P[f"l{ln}b0_c1_w"], P[f"l{ln}b0_c1_b"], relu=True)
            y1 = y1.reshape(nb, hh, ww, -1)
            y2 = _conv3x3_s2(y1, P[f"l{ln}b0_c2_w"], P[f"l{ln}b0_c2_b"])
            xs = h[:, ::2, ::2, :]
            no, ho2, wo2, _ = y2.shape
            out = _matmul(y2.reshape(no * ho2 * wo2, -1),
                          P[f"l{ln}b0_c3_w"], P[f"l{ln}b0_c3_b"], relu=True,
                          r=xs.reshape(no * ho2 * wo2, C),
                          wd=P[f"l{ln}b0_down_w"], bd=P[f"l{ln}b0_down_b"])
            h = out.reshape(no, ho2, wo2, -1)
            first_fused = 1
        else:
            first_fused = 0

        nb, hh, ww, C = h.shape
        Cin_needed = P[f"l{ln}b{first_fused}_c1_w"].shape[0]
        hflat, hp, wp = _pad_flat(h, Cin_needed)
        H = max(16, _np2(wp + 2))
        mask = _ring_mask(nb, hp, wp, H, 256)
        for b in range(first_fused, blocks):
            if b == 0:
                hflat = _fused_block(hflat, mask, nb, hp, wp,
                                     P[f"l{ln}b0_c1_w"], P[f"l{ln}b0_c1_b"],
                                     P[f"l{ln}b0_c2_w"], P[f"l{ln}b0_c2_b"],
                                     P[f"l{ln}b0_c3_w"], P[f"l{ln}b0_c3_b"],
                                     wd=P[f"l{ln}b0_down_w"],
                                     bd=P[f"l{ln}b0_down_b"])
            else:
                hflat = _fused_block(hflat, mask, nb, hp, wp,
                                     P[f"l{ln}b{b}_c1_w"], P[f"l{ln}b{b}_c1_b"],
                                     P[f"l{ln}b{b}_c2_w"], P[f"l{ln}b{b}_c2_b"],
                                     P[f"l{ln}b{b}_c3_w"], P[f"l{ln}b{b}_c3_b"])
        h = _unflat(hflat, nb, hp, wp)

    feat = _gap(h)
    logits = _matmul(feat, head_w, head_b, relu=False)
    return logits[:, :1000].astype(jnp.float32)


# speedup vs baseline: 1.2092x; 1.0432x over previous
import functools

import jax
import jax.numpy as jnp
from jax.experimental import pallas as pl
from jax.experimental.pallas import tpu as pltpu


def _rup(x, m):
    return ((x + m - 1) // m) * m


def _np2(x):
    n = 1
    while n < x:
        n *= 2
    return n


_VMEM = 64 * 1024 * 1024


def _mm_body(*refs, relu, mode):
    if mode == "plain":
        x_ref, w_ref, b_ref, o_ref = refs
        y = jnp.dot(x_ref[...], w_ref[...], preferred_element_type=jnp.float32)
        y = y + b_ref[...]
    else:
        x_ref, w_ref, b_ref, r_ref, wd_ref, bd_ref, o_ref = refs
        y = jnp.dot(x_ref[...], w_ref[...], preferred_element_type=jnp.float32)
        y = y + b_ref[...]
        y = y + jnp.dot(r_ref[...], wd_ref[...], preferred_element_type=jnp.float32)
        y = y + bd_ref[...]
    if relu:
        y = jnp.maximum(y, 0.0)
    o_ref[...] = y.astype(o_ref.dtype)


def _matmul(x, w, b, relu, r=None, wd=None, bd=None):
    M, K = x.shape
    N = w.shape[1]
    tm = min(512, _rup(M, 16))
    Mp = _rup(M, tm)
    tn = min(512, N)
    if Mp != M:
        x = jnp.pad(x, ((0, Mp - M), (0, 0)))
        if r is not None:
            r = jnp.pad(r, ((0, Mp - M), (0, 0)))
    mode = "plain" if r is None else "proj"
    inputs = [x, w, b]
    specs = [
        pl.BlockSpec((tm, K), lambda i, j: (i, 0)),
        pl.BlockSpec((K, tn), lambda i, j: (0, j)),
        pl.BlockSpec((1, tn), lambda i, j: (0, j)),
    ]
    if r is not None:
        Kd = r.shape[1]
        inputs += [r, wd, bd]
        specs += [
            pl.BlockSpec((tm, Kd), lambda i, j: (i, 0)),
            pl.BlockSpec((Kd, tn), lambda i, j: (0, j)),
            pl.BlockSpec((1, tn), lambda i, j: (0, j)),
        ]
    out = pl.pallas_call(
        functools.partial(_mm_body, relu=relu, mode=mode),
        out_shape=jax.ShapeDtypeStruct((Mp, N), jnp.bfloat16),
        grid=(Mp // tm, N // tn),
        in_specs=specs,
        out_specs=pl.BlockSpec((tm, tn), lambda i, j: (i, j)),
        compiler_params=pltpu.CompilerParams(
            dimension_semantics=("parallel", "parallel"),
            vmem_limit_bytes=_VMEM),
    )(*inputs)
    return out if Mp == M else out[:M]


def _block_kernel(refs, *, wp, H, tm, Cm, down):
    if down:
        (xm_ref, xh_ref, mm_ref, mh_ref, w1_ref, b1_ref, w2_ref, b2_ref,
         w3_ref, b3_ref, wd_ref, bd_ref, o_ref, y1_ref) = refs
    else:
        (xm_ref, xh_ref, mm_ref, mh_ref, w1_ref, b1_ref, w2_ref, b2_ref,
         w3_ref, b3_ref, o_ref, y1_ref) = refs
        wd_ref = bd_ref = None
    y1m = jnp.dot(xm_ref[...], w1_ref[...], preferred_element_type=jnp.float32)
    y1m = jnp.maximum(y1m + b1_ref[...], 0.0) * mm_ref[...]
    y1_ref[0:tm, :] = y1m
    y1h = jnp.dot(xh_ref[...], w1_ref[...], preferred_element_type=jnp.float32)
    y1h = jnp.maximum(y1h + b1_ref[...], 0.0) * mh_ref[...]
    y1_ref[tm:, :] = y1h
    base = H - (wp + 1)
    acc = b2_ref[...].astype(jnp.float32)
    for t in range(9):
        off = base + (t // 3) * wp + (t % 3)
        tap = y1_ref[off:off + tm, :].astype(jnp.bfloat16)
        acc = acc + jnp.dot(tap, w2_ref[t * Cm:(t + 1) * Cm, :],
                            preferred_element_type=jnp.float32)
    y2 = jnp.maximum(acc, 0.0).astype(jnp.bfloat16)
    y3 = jnp.dot(y2, w3_ref[...], preferred_element_type=jnp.float32)
    y3 = y3 + b3_ref[...]
    res = jnp.concatenate([xm_ref[H:tm, :], xh_ref[0:H, :]], axis=0)
    if down:
        rp = jnp.dot(res, wd_ref[...], preferred_element_type=jnp.float32)
        y3 = y3 + rp + bd_ref[...]
    else:
        y3 = y3 + res.astype(jnp.float32)
    omask = jnp.concatenate([mm_ref[H:tm, :], mh_ref[0:H, :]], axis=0)
    o_ref[...] = (jnp.maximum(y3, 0.0) * omask).astype(o_ref.dtype)


def _fused_block(xflat, maskext, n, hp, wp, w1, b1, w2, b2, w3, b3,
                 wd=None, bd=None):
    M = n * hp * wp
    Cin = xflat.shape[1]
    Cm = w2.shape[1]
    Cout = w3.shape[1]
    H = max(16, _np2(wp + 2))
    tm = 512
    Mp = _rup(M, tm)
    xext = jnp.pad(xflat, ((H, Mp - M + H), (0, 0)))
    nhb = tm // (2 * H)

    refs_in = [xext, xext, maskext, maskext, w1, b1, w2, b2, w3, b3]
    specs = [
        pl.BlockSpec((tm, Cin), lambda i: (i, 0)),
        pl.BlockSpec((2 * H, Cin), lambda i: ((i + 1) * nhb, 0)),
        pl.BlockSpec((tm, 1), lambda i: (i, 0)),
        pl.BlockSpec((2 * H, 1), lambda i: ((i + 1) * nhb, 0)),
        pl.BlockSpec((Cin, Cm), lambda i: (0, 0)),
        pl.BlockSpec((1, Cm), lambda i: (0, 0)),
        pl.BlockSpec((9 * Cm, Cm), lambda i: (0, 0)),
        pl.BlockSpec((1, Cm), lambda i: (0, 0)),
        pl.BlockSpec((Cm, Cout), lambda i: (0, 0)),
        pl.BlockSpec((1, Cout), lambda i: (0, 0)),
    ]
    down = wd is not None
    if down:
        refs_in += [wd, bd]
        specs += [
            pl.BlockSpec((Cin, Cout), lambda i: (0, 0)),
            pl.BlockSpec((1, Cout), lambda i: (0, 0)),
        ]

    def body(*refs):
        _block_kernel(refs, wp=wp, H=H, tm=tm, Cm=Cm, down=down)

    out = pl.pallas_call(
        body,
        out_shape=jax.ShapeDtypeStruct((Mp, Cout), jnp.bfloat16),
        grid=(Mp // tm,),
        in_specs=specs,
        out_specs=pl.BlockSpec((tm, Cout), lambda i: (i, 0)),
        scratch_shapes=[pltpu.VMEM((tm + 2 * H, Cm), jnp.float32)],
        compiler_params=pltpu.CompilerParams(
            dimension_semantics=("parallel",),
            vmem_limit_bytes=_VMEM),
    )(*refs_in)
    return out[:M]


def _ring_mask(n, hp, wp, H, tm):
    M = n * hp * wp
    Mp = _rup(M, tm)
    r = jnp.arange(M, dtype=jnp.int32)
    p = r % (hp * wp)
    a = p // wp
    b = p % wp
    ok = (a >= 1) & (a < hp - 1) & (b >= 1) & (b < wp - 1)
    m = ok.astype(jnp.float32).reshape(M, 1)
    return jnp.pad(m, ((H, Mp - M + H), (0, 0)))


def _s2conv_body(zm_ref, zh_ref, w_ref, b_ref, o_ref, zbuf_ref, *, Wv, tm, C4):
    tm_, _ = zm_ref.shape
    zbuf_ref[0:tm, :] = zm_ref[...].astype(jnp.float32)
    zbuf_ref[tm:, :] = zh_ref[...].astype(jnp.float32)
    acc = b_ref[...].astype(jnp.float32)
    for t, off in enumerate((0, 1, Wv, Wv + 1)):
        tap = zbuf_ref[off:off + tm, :].astype(jnp.bfloat16)
        acc = acc + jnp.dot(tap, w_ref[t * C4:(t + 1) * C4, :],
                            preferred_element_type=jnp.float32)
    o_ref[...] = jnp.maximum(acc, 0.0).astype(o_ref.dtype)


def _conv3x3_s2(x, w, b):
    n, h, wdt, C = x.shape
    xpad = jnp.pad(x, ((0, 0), (1, 1), (1, 1), (0, 0)))
    Ha, Wv = (h + 2) // 2, (wdt + 2) // 2
    z = xpad.reshape(n, Ha, 2, Wv, 2, C)
    z = jnp.transpose(z, (0, 1, 3, 2, 4, 5)).reshape(n * Ha * Wv, 4 * C)
    M = n * Ha * Wv
    H = max(16, _np2(Wv + 2))
    tm = 256
    Mp = _rup(M, tm)
    zext = jnp.pad(z, ((0, Mp - M + H), (0, 0)))
    Cout = w.shape[1]
    out = pl.pallas_call(
        functools.partial(_s2conv_body, Wv=Wv, tm=tm, C4=4 * C),
        out_shape=jax.ShapeDtypeStruct((Mp, Cout), jnp.bfloat16),
        grid=(Mp // tm,),
        in_specs=[
            pl.BlockSpec((tm, 4 * C), lambda i: (i, 0)),
            pl.BlockSpec((H, 4 * C), lambda i: ((i + 1) * (tm // H), 0)),
            pl.BlockSpec((16 * C, Cout), lambda i: (0, 0)),
            pl.BlockSpec((1, Cout), lambda i: (0, 0)),
        ],
        out_specs=pl.BlockSpec((tm, Cout), lambda i: (i, 0)),
        scratch_shapes=[pltpu.VMEM((tm + H, 4 * C), jnp.float32)],
        compiler_params=pltpu.CompilerParams(
            dimension_semantics=("parallel",),
            vmem_limit_bytes=_VMEM),
    )(zext, zext, w, b)
    ho, wo = h // 2, wdt // 2
    return out[:M].reshape(n, Ha, Wv, Cout)[:, :ho, :wo, :]


def _maxpool_body(x_ref, o_ref, *, ho, wo, c):
    X = x_ref[0].astype(jnp.float32)
    hp = X.shape[0]
    E = X.reshape(hp // 2, 2, X.shape[1], 2 * c)
    ev = E[:, 0]
    od = E[:, 1]
    v = jnp.maximum(jnp.maximum(ev[0:ho], od[0:ho]), ev[1:ho + 1])
    m = jnp.maximum(jnp.maximum(v[:, 0:wo, 0:c], v[:, 0:wo, c:2 * c]),
                    v[:, 1:wo + 1, 0:c])
    o_ref[0] = m.astype(o_ref.dtype)


def _maxpool(x):
    n, h, w, c = x.shape
    ho, wo = (h + 2 - 3) // 2 + 1, (w + 2 - 3) // 2 + 1
    xp = jnp.pad(x, ((0, 0), (1, 1), (1, 1), (0, 0)))
    hp, wv = h + 2, (w + 2) // 2
    xv = xp.reshape(n, hp, wv, 2 * c)
    return pl.pallas_call(
        functools.partial(_maxpool_body, ho=ho, wo=wo, c=c),
        out_shape=jax.ShapeDtypeStruct((n, ho, wo, c), x.dtype),
        grid=(n,),
        in_specs=[pl.BlockSpec((1, hp, wv, 2 * c), lambda i: (i, 0, 0, 0))],
        out_specs=pl.BlockSpec((1, ho, wo, c), lambda i: (i, 0, 0, 0)),
        compiler_params=pltpu.CompilerParams(
            dimension_semantics=("parallel",),
            vmem_limit_bytes=_VMEM),
    )(xv)


def _gap_body(x_ref, o_ref, *, inv):
    o_ref[...] = (jnp.sum(x_ref[...].astype(jnp.float32), axis=1,
                          keepdims=True) * inv).astype(o_ref.dtype)


def _gap(x):
    n, h, w, c = x.shape
    x2 = x.reshape(n, h * w, c)
    out = pl.pallas_call(
        functools.partial(_gap_body, inv=1.0 / (h * w)),
        out_shape=jax.ShapeDtypeStruct((n, 1, c), x.dtype),
        grid=(n,),
        in_specs=[pl.BlockSpec((1, h * w, c), lambda i: (i, 0, 0))],
        out_specs=pl.BlockSpec((1, 1, c), lambda i: (i, 0, 0)),
        compiler_params=pltpu.CompilerParams(
            dimension_semantics=("parallel",),
            vmem_limit_bytes=_VMEM),
    )(x2)
    return out.reshape(n, c)


def _stem(x, w, b):
    n, h, wdt, c = x.shape
    xp = jnp.pad(x, ((0, 0), (3, 3), (3, 3), (0, 0)))
    ho = (h + 6 - 7) // 2 + 1
    wo = (wdt + 6 - 7) // 2 + 1
    cols = [xp[:, i:i + 2 * ho:2, j:j + 2 * wo:2, :]
            for i in range(7) for j in range(7)]
    patches = jnp.stack(cols, axis=3).reshape(n * ho * wo, 49 * c)
    Kp = w.shape[0]
    patches = jnp.pad(patches, ((0, 0), (0, Kp - 49 * c)))
    out = _matmul(patches, w, b, relu=True)
    return out.reshape(n, ho, wo, -1)


_CFG = [(3, 1), (4, 2), (6, 2), (3, 2)]


def _pad_flat(x, cpad):
    n, h, w, c = x.shape
    xp = jnp.pad(x, ((0, 0), (1, 1), (1, 1), (0, cpad - c)))
    return xp.reshape(n * (h + 2) * (w + 2), cpad), h + 2, w + 2


def _unflat(xflat, n, hp, wp):
    return xflat.reshape(n, hp, wp, -1)[:, 1:hp - 1, 1:wp - 1, :]


def kernel(stem_w, stem_b, l1b0_c1_w, l1b0_c1_b, l1b0_c2_w, l1b0_c2_b, l1b0_c3_w, l1b0_c3_b, l1b0_down_w, l1b0_down_b, l1b1_c1_w, l1b1_c1_b, l1b1_c2_w, l1b1_c2_b, l1b1_c3_w, l1b1_c3_b, l1b2_c1_w, l1b2_c1_b, l1b2_c2_w, l1b2_c2_b, l1b2_c3_w, l1b2_c3_b, l2b0_c1_w, l2b0_c1_b, l2b0_c2_w, l2b0_c2_b, l2b0_c3_w, l2b0_c3_b, l2b0_down_w, l2b0_down_b, l2b1_c1_w, l2b1_c1_b, l2b1_c2_w, l2b1_c2_b, l2b1_c3_w, l2b1_c3_b, l2b2_c1_w, l2b2_c1_b, l2b2_c2_w, l2b2_c2_b, l2b2_c3_w, l2b2_c3_b, l2b3_c1_w, l2b3_c1_b, l2b3_c2_w, l2b3_c2_b, l2b3_c3_w, l2b3_c3_b, l3b0_c1_w, l3b0_c1_b, l3b0_c2_w, l3b0_c2_b, l3b0_c3_w, l3b0_c3_b, l3b0_down_w, l3b0_down_b, l3b1_c1_w, l3b1_c1_b, l3b1_c2_w, l3b1_c2_b, l3b1_c3_w, l3b1_c3_b, l3b2_c1_w, l3b2_c1_b, l3b2_c2_w, l3b2_c2_b, l3b2_c3_w, l3b2_c3_b, l3b3_c1_w, l3b3_c1_b, l3b3_c2_w, l3b3_c2_b, l3b3_c3_w, l3b3_c3_b, l3b4_c1_w, l3b4_c1_b, l3b4_c2_w, l3b4_c2_b, l3b4_c3_w, l3b4_c3_b, l3b5_c1_w, l3b5_c1_b, l3b5_c2_w, l3b5_c2_b, l3b5_c3_w, l3b5_c3_b, l4b0_c1_w, l4b0_c1_b, l4b0_c2_w, l4b0_c2_b, l4b0_c3_w, l4b0_c3_b, l4b0_down_w, l4b0_down_b, l4b1_c1_w, l4b1_c1_b, l4b1_c2_w, l4b1_c2_b, l4b1_c3_w, l4b1_c3_b, l4b2_c1_w, l4b2_c1_b, l4b2_c2_w, l4b2_c2_b, l4b2_c3_w, l4b2_c3_b, head_w, head_b, x):
    P = locals()
    n = x.shape[0]
    h = jnp.transpose(x, (0, 2, 3, 1)).astype(jnp.bfloat16)
    h = _stem(h, stem_w, stem_b)
    h = _maxpool(h)

    for li, (blocks, stride) in enumerate(_CFG):
        ln = li + 1
        if stride == 2:
            nb, hh, ww, C = h.shape
            y1 = _matmul(h.reshape(nb * hh * ww, C),
                         P[f"l{ln}b0_c1_w"], P[f"l{ln}b0_c1_b"], relu=True)
            y1 = y1.reshape(nb, hh, ww, -1)
            y2 = _conv3x3_s2(y1, P[f"l{ln}b0_c2_w"], P[f"l{ln}b0_c2_b"])
            xs = h[:, ::2, ::2, :]
            no, ho2, wo2, _ = y2.shape
            out = _matmul(y2.reshape(no * ho2 * wo2, -1),
                          P[f"l{ln}b0_c3_w"], P[f"l{ln}b0_c3_b"], relu=True,
                          r=xs.reshape(no * ho2 * wo2, C),
                          wd=P[f"l{ln}b0_down_w"], bd=P[f"l{ln}b0_down_b"])
            h = out.reshape(no, ho2, wo2, -1)
            first_fused = 1
        else:
            first_fused = 0

        nb, hh, ww, C = h.shape
        Cin_needed = P[f"l{ln}b{first_fused}_c1_w"].shape[0]
        hflat, hp, wp = _pad_flat(h, Cin_needed)
        H = max(16, _np2(wp + 2))
        mask = _ring_mask(nb, hp, wp, H, 512)
        for b in range(first_fused, blocks):
            if b == 0:
                hflat = _fused_block(hflat, mask, nb, hp, wp,
                                     P[f"l{ln}b0_c1_w"], P[f"l{ln}b0_c1_b"],
                                     P[f"l{ln}b0_c2_w"], P[f"l{ln}b0_c2_b"],
                                     P[f"l{ln}b0_c3_w"], P[f"l{ln}b0_c3_b"],
                                     wd=P[f"l{ln}b0_down_w"],
                                     bd=P[f"l{ln}b0_down_b"])
            else:
                hflat = _fused_block(hflat, mask, nb, hp, wp,
                                     P[f"l{ln}b{b}_c1_w"], P[f"l{ln}b{b}_c1_b"],
                                     P[f"l{ln}b{b}_c2_w"], P[f"l{ln}b{b}_c2_b"],
                                     P[f"l{ln}b{b}_c3_w"], P[f"l{ln}b{b}_c3_b"])
        h = _unflat(hflat, nb, hp, wp)

    feat = _gap(h)
    logits = _matmul(feat, head_w, head_b, relu=False)
    return logits[:, :1000].astype(jnp.float32)
